# baseline (speedup 1.0000x reference)
import functools

import jax
import jax.numpy as jnp
from jax import lax
from jax.experimental import pallas as pl
from jax.experimental.pallas import tpu as pltpu
from jax.experimental.pallas import tpu_sc as plsc

NUM_CORES = 2
NUM_SUBCORES = 16
NW = NUM_CORES * NUM_SUBCORES
BATCH = 16384
B_PER_W = BATCH // NW
EMBED = 32
LANES = 16

SEG_W = 31232
CHUNK_W = 256
NCH_REG = SEG_W // CHUNK_W
GCAP = 1552
LCAP = 288
DUMMY_C = 0x3FFFFF00
DUMMY_B = BATCH
NVEC_IDX = BATCH // LANES
ROWCAP = 4096
POS_N = BATCH + LANES


def _memb_bounds(c, seg_lo):
    reg = seg_lo + c * CHUNK_W
    fetch = jnp.where(c < NCH_REG, reg,
                      jnp.where(c == NCH_REG, 999424,
                                jnp.where(c == NCH_REG + 1, 999680, 999808)))
    lo = jnp.where(c < NCH_REG, fetch,
                   jnp.where(c < NCH_REG + 2, fetch, 999936))
    hi = jnp.where(c < NCH_REG + 2, lo + CHUNK_W, 1000000)
    return fetch, lo, hi


def _issue_chunk(tab, fetch, buf, par, sem):
    pltpu.async_copy(tab.at[:, pl.ds(pl.multiple_of(fetch, 128), CHUNK_W)],
                     buf.at[par], sem)


def _wait_chunk(tab, fetch, buf, par, sem):
    pltpu.make_async_copy(tab.at[:, pl.ds(pl.multiple_of(fetch, 128), CHUNK_W)],
                          buf.at[par], sem).wait()


def _phase_a_body(user_hbm, item_hbm, ut_t, it_t,
                  stag_u, stag_i, posmap_u, posmap_i,
                  uidx_all, iidx_all, gcu, gbu, gcv, gbv, lc, lb,
                  cbuf_u, cbuf_v, flush_u, flush_v, posb,
                  sem_s, sem_fu, sem_fv, sem_p):
    core = lax.axis_index("c")
    sub = lax.axis_index("s")
    wid = sub * NUM_CORES + core
    seg_lo = wid * SEG_W
    seg_hi = seg_lo + SEG_W
    is_t0 = wid == 0
    nch = NCH_REG + jnp.where(is_t0, 3, 0)
    region = wid * ROWCAP

    pltpu.sync_copy(user_hbm, uidx_all)
    pltpu.sync_copy(item_hbm, iidx_all)

    for c0 in range(2):
        f0, _, _ = _memb_bounds(jnp.int32(c0), seg_lo)
        _issue_chunk(ut_t, f0, cbuf_u, c0, sem_s)
        _issue_chunk(it_t, f0, cbuf_v, c0, sem_s)

    iota = lax.iota(jnp.int32, LANES)

    def filt(i, carry):
        nu, nv = carry
        bv = i * LANES + iota
        uv = uidx_all[pl.ds(i * LANES, LANES)]
        m = (uv >= seg_lo) & (uv < seg_hi)
        m = m | ((uv >= 999424) & jnp.full((LANES,), is_t0))
        plsc.store_compressed(gcu.at[pl.ds(nu, LANES)], uv, mask=m)
        plsc.store_compressed(gbu.at[pl.ds(nu, LANES)], bv, mask=m)
        nu = nu + plsc.all_reduce_population_count(m)[0]
        vv = iidx_all[pl.ds(i * LANES, LANES)]
        m2 = (vv >= seg_lo) & (vv < seg_hi)
        m2 = m2 | ((vv >= 999424) & jnp.full((LANES,), is_t0))
        plsc.store_compressed(gcv.at[pl.ds(nv, LANES)], vv, mask=m2)
        plsc.store_compressed(gbv.at[pl.ds(nv, LANES)], bv, mask=m2)
        nv = nv + plsc.all_reduce_population_count(m2)[0]
        return nu, nv

    nu, nv = lax.fori_loop(0, NVEC_IDX, filt, (jnp.int32(0), jnp.int32(0)))
    gcu[pl.ds(nu, LANES)] = jnp.full((LANES,), DUMMY_C, jnp.int32)
    gbu[pl.ds(nu, LANES)] = jnp.full((LANES,), DUMMY_B, jnp.int32)
    gcv[pl.ds(nv, LANES)] = jnp.full((LANES,), DUMMY_C, jnp.int32)
    gbv[pl.ds(nv, LANES)] = jnp.full((LANES,), DUMMY_B, jnp.int32)
    nvec_u = (nu + 2 * LANES - 1) >> 4
    nvec_v = (nv + 2 * LANES - 1) >> 4

    def process(cbuf, par, gc, gb, nvec, fetch, mlo, mhi,
                stag, posmap, flush, sem_f, gcnt):
        def memb(i2, lcnt):
            cv2 = gc[pl.ds(i2 * LANES, LANES)]
            bv2 = gb[pl.ds(i2 * LANES, LANES)]
            m = (cv2 >= mlo) & (cv2 < mhi)
            plsc.store_compressed(lc.at[pl.ds(lcnt, LANES)], cv2, mask=m)
            plsc.store_compressed(lb.at[pl.ds(lcnt, LANES)], bv2, mask=m)
            return lcnt + plsc.all_reduce_population_count(m)[0]

        lcnt = lax.fori_loop(0, nvec, memb, jnp.int32(0))
        lc[pl.ds(lcnt, LANES)] = jnp.full((LANES,), fetch, jnp.int32)
        lb[pl.ds(lcnt, LANES)] = jnp.full((LANES,), DUMMY_B, jnp.int32)
        ngr = (lcnt + LANES - 1) >> 4

        def group(g2, gc2):
            cvec = lc[pl.ds(g2 * LANES, LANES)]
            bvec = lb[pl.ds(g2 * LANES, LANES)]
            lvec = cvec - fetch
            slot = gc2 & 1
            wpos = region + gc2 * LANES
            posb[slot, pl.ds(0, LANES)] = wpos + iota
            hp = pltpu.async_copy(posb.at[slot], posmap.at[bvec], sem_p)

            @pl.when(gc2 >= 2)
            def _():
                pltpu.make_async_copy(
                    flush.at[slot],
                    stag.at[pl.ds(region + (gc2 - 2) * LANES, LANES)],
                    sem_f).wait()

            rows = iota
            for j in range(LANES):
                cols = jnp.full((LANES,), lvec[j], jnp.int32)
                lo = plsc.load_gather(cbuf.at[par], [rows, cols])
                hi = plsc.load_gather(cbuf.at[par], [rows + LANES, cols])
                flush[slot, j, pl.ds(0, LANES)] = lo
                flush[slot, j, pl.ds(LANES, LANES)] = hi
            hp.wait()
            pltpu.async_copy(flush.at[slot], stag.at[pl.ds(wpos, LANES)],
                             sem_f)
            return gc2 + 1

        return lax.fori_loop(0, ngr, group, gcnt)

    def chunk_body(c, carry):
        gu, gv = carry
        par = lax.rem(c, 2)
        fetch, mlo, mhi = _memb_bounds(c, seg_lo)
        _wait_chunk(ut_t, fetch, cbuf_u, par, sem_s)
        _wait_chunk(it_t, fetch, cbuf_v, par, sem_s)
        gu = process(cbuf_u, par, gcu, gbu, nvec_u, fetch, mlo, mhi,
                     stag_u, posmap_u, flush_u, sem_fu, gu)
        gv = process(cbuf_v, par, gcv, gbv, nvec_v, fetch, mlo, mhi,
                     stag_i, posmap_i, flush_v, sem_fv, gv)

        @pl.when(c + 2 < nch)
        def _():
            f2, _, _ = _memb_bounds(c + 2, seg_lo)
            _issue_chunk(ut_t, f2, cbuf_u, par, sem_s)
            _issue_chunk(it_t, f2, cbuf_v, par, sem_s)

        return gu, gv

    gu, gv = lax.fori_loop(0, nch, chunk_body, (jnp.int32(0), jnp.int32(0)))

    for gcnt, flush, stag, sem_f in ((gu, flush_u, stag_u, sem_fu),
                                     (gv, flush_v, stag_i, sem_fv)):
        @pl.when(gcnt >= 2)
        def _(gcnt=gcnt, flush=flush, stag=stag, sem_f=sem_f):
            pltpu.make_async_copy(
                flush.at[gcnt & 1],
                stag.at[pl.ds(region + (gcnt - 2) * LANES, LANES)],
                sem_f).wait()

        @pl.when(gcnt >= 1)
        def _(gcnt=gcnt, flush=flush, stag=stag, sem_f=sem_f):
            pltpu.make_async_copy(
                flush.at[(gcnt - 1) & 1],
                stag.at[pl.ds(region + (gcnt - 1) * LANES, LANES)],
                sem_f).wait()


@functools.partial(
    pl.kernel,
    out_type=(jax.ShapeDtypeStruct((NW * ROWCAP, 128), jnp.float32),
              jax.ShapeDtypeStruct((NW * ROWCAP, 128), jnp.float32),
              jax.ShapeDtypeStruct((POS_N,), jnp.int32),
              jax.ShapeDtypeStruct((POS_N,), jnp.int32)),
    mesh=plsc.VectorSubcoreMesh(core_axis_name="c", subcore_axis_name="s"),
    scratch_types=[
        pltpu.VMEM((BATCH,), jnp.int32),
        pltpu.VMEM((BATCH,), jnp.int32),
        pltpu.VMEM((GCAP,), jnp.int32),
        pltpu.VMEM((GCAP,), jnp.int32),
        pltpu.VMEM((GCAP,), jnp.int32),
        pltpu.VMEM((GCAP,), jnp.int32),
        pltpu.VMEM((LCAP,), jnp.int32),
        pltpu.VMEM((LCAP,), jnp.int32),
        pltpu.VMEM((2, EMBED, CHUNK_W), jnp.float32),
        pltpu.VMEM((2, EMBED, CHUNK_W), jnp.float32),
        pltpu.VMEM((2, LANES, 128), jnp.float32),
        pltpu.VMEM((2, LANES, 128), jnp.float32),
        pltpu.VMEM((2, LANES), jnp.int32),
        pltpu.SemaphoreType.DMA,
        pltpu.SemaphoreType.DMA,
        pltpu.SemaphoreType.DMA,
        pltpu.SemaphoreType.DMA,
    ],
    compiler_params=pltpu.CompilerParams(needs_layout_passes=False),
)
def _phase_a(user_hbm, item_hbm, ut_t, it_t,
             stag_u, stag_i, posmap_u, posmap_i, *scratch):
    _phase_a_body(user_hbm, item_hbm, ut_t, it_t,
                  stag_u, stag_i, posmap_u, posmap_i, *scratch)


RB_CH = 4
RB_W = B_PER_W // RB_CH


def _phase_b_body(stag_u, stag_i, posmap_u, posmap_i, out_hbm,
                  pos_u, pos_v, sbu, sbv, out_v, sem):
    core = lax.axis_index("c")
    sub = lax.axis_index("s")
    wid = sub * NUM_CORES + core
    base = wid * B_PER_W

    pltpu.sync_copy(posmap_u.at[pl.ds(base, B_PER_W)], pos_u)
    pltpu.sync_copy(posmap_i.at[pl.ds(base, B_PER_W)], pos_v)

    handles = [None] * (RB_CH + 1)
    handles[0] = (
        pltpu.async_copy(stag_u.at[pos_u.at[pl.ds(0, RB_W)]], sbu.at[0], sem),
        pltpu.async_copy(stag_i.at[pos_v.at[pl.ds(0, RB_W)]], sbv.at[0], sem),
    )
    last_lane = lax.iota(jnp.int32, LANES) == (LANES - 1)

    for c in range(RB_CH):
        par = c % 2
        if c + 1 < RB_CH:
            nxt = (c + 1) % 2
            handles[c + 1] = (
                pltpu.async_copy(
                    stag_u.at[pos_u.at[pl.ds((c + 1) * RB_W, RB_W)]],
                    sbu.at[nxt], sem),
                pltpu.async_copy(
                    stag_i.at[pos_v.at[pl.ds((c + 1) * RB_W, RB_W)]],
                    sbv.at[nxt], sem),
            )
        hu, hv = handles[c]
        hu.wait()
        hv.wait()

        def row(r, carry2, c=c, par=par):
            u0 = sbu[par, r, pl.ds(0, LANES)]
            u1 = sbu[par, r, pl.ds(LANES, LANES)]
            v0 = sbv[par, r, pl.ds(0, LANES)]
            v1 = sbv[par, r, pl.ds(LANES, LANES)]
            cs = plsc.cumsum(u0 * v0 + u1 * v1)
            plsc.store_scatter(out_v,
                               [jnp.full((LANES,), c * RB_W + r, jnp.int32)],
                               cs, mask=last_lane)
            return carry2

        lax.fori_loop(0, RB_W, row, 0)

    pltpu.sync_copy(out_v, out_hbm.at[pl.ds(base, B_PER_W)])


@functools.partial(
    pl.kernel,
    out_type=jax.ShapeDtypeStruct((BATCH,), jnp.float32),
    mesh=plsc.VectorSubcoreMesh(core_axis_name="c", subcore_axis_name="s"),
    scratch_types=[
        pltpu.VMEM((B_PER_W,), jnp.int32),
        pltpu.VMEM((B_PER_W,), jnp.int32),
        pltpu.VMEM((2, RB_W, 128), jnp.float32),
        pltpu.VMEM((2, RB_W, 128), jnp.float32),
        pltpu.VMEM((B_PER_W,), jnp.float32),
        pltpu.SemaphoreType.DMA,
    ],
    compiler_params=pltpu.CompilerParams(needs_layout_passes=False),
)
def _phase_b(stag_u, stag_i, posmap_u, posmap_i, out_hbm, *scratch):
    _phase_b_body(stag_u, stag_i, posmap_u, posmap_i, out_hbm, *scratch)


def kernel(user, item, user_table, item_table):
    su, si, pu, pv = _phase_a(user.astype(jnp.int32), item.astype(jnp.int32),
                              user_table.T, item_table.T)
    return _phase_b(su, si, pu, pv)


# speedup vs baseline: 1.0057x; 1.0057x over previous
import functools

import jax
import jax.numpy as jnp
from jax import lax
from jax.experimental import pallas as pl
from jax.experimental.pallas import tpu as pltpu
from jax.experimental.pallas import tpu_sc as plsc

NUM_CORES = 2
NUM_SUBCORES = 16
NW = NUM_CORES * NUM_SUBCORES
BATCH = 16384
B_PER_W = BATCH // NW
EMBED = 32
LANES = 16

SEG_W = 31232
CHUNK_W = 256
NCH_REG = SEG_W // CHUNK_W
GCAP = 1552
LCAP = 288
DUMMY_C = 0x3FFFFF00
DUMMY_B = BATCH
NVEC_IDX = BATCH // LANES
ROWCAP = 4096
POS_N = BATCH + LANES


def _memb_bounds(c, seg_lo):
    reg = seg_lo + c * CHUNK_W
    fetch = jnp.where(c < NCH_REG, reg,
                      jnp.where(c == NCH_REG, 999424,
                                jnp.where(c == NCH_REG + 1, 999680, 999808)))
    lo = jnp.where(c < NCH_REG, fetch,
                   jnp.where(c < NCH_REG + 2, fetch, 999936))
    hi = jnp.where(c < NCH_REG + 2, lo + CHUNK_W, 1000000)
    return fetch, lo, hi


def _issue_chunk(tab, fetch, buf, par, sem):
    pltpu.async_copy(tab.at[:, pl.ds(pl.multiple_of(fetch, 128), CHUNK_W)],
                     buf.at[par], sem)


def _wait_chunk(tab, fetch, buf, par, sem):
    pltpu.make_async_copy(tab.at[:, pl.ds(pl.multiple_of(fetch, 128), CHUNK_W)],
                          buf.at[par], sem).wait()


def _phase_a_body(user_hbm, item_hbm, ut_t, it_t,
                  stag_u, stag_i, posmap_u, posmap_i,
                  uidx_all, iidx_all, gcu, gbu, gcv, gbv, lc, lb,
                  cbuf_u, cbuf_v, flush_u, flush_v, posb,
                  sem_s, sem_fu, sem_fv, sem_p):
    core = lax.axis_index("c")
    sub = lax.axis_index("s")
    wid = sub * NUM_CORES + core
    seg_lo = wid * SEG_W
    seg_hi = seg_lo + SEG_W
    is_t0 = wid == 0
    nch = NCH_REG + jnp.where(is_t0, 3, 0)
    region = wid * ROWCAP

    pltpu.sync_copy(user_hbm, uidx_all)
    pltpu.sync_copy(item_hbm, iidx_all)

    for c0 in range(2):
        f0, _, _ = _memb_bounds(jnp.int32(c0), seg_lo)
        _issue_chunk(ut_t, f0, cbuf_u, c0, sem_s)
        _issue_chunk(it_t, f0, cbuf_v, c0, sem_s)

    iota = lax.iota(jnp.int32, LANES)

    def filt(i, carry):
        nu, nv = carry
        bv = i * LANES + iota
        uv = uidx_all[pl.ds(i * LANES, LANES)]
        m = (uv >= seg_lo) & (uv < seg_hi)
        m = m | ((uv >= 999424) & jnp.full((LANES,), is_t0))
        plsc.store_compressed(gcu.at[pl.ds(nu, LANES)], uv, mask=m)
        plsc.store_compressed(gbu.at[pl.ds(nu, LANES)], bv, mask=m)
        nu = nu + plsc.all_reduce_population_count(m)[0]
        vv = iidx_all[pl.ds(i * LANES, LANES)]
        m2 = (vv >= seg_lo) & (vv < seg_hi)
        m2 = m2 | ((vv >= 999424) & jnp.full((LANES,), is_t0))
        plsc.store_compressed(gcv.at[pl.ds(nv, LANES)], vv, mask=m2)
        plsc.store_compressed(gbv.at[pl.ds(nv, LANES)], bv, mask=m2)
        nv = nv + plsc.all_reduce_population_count(m2)[0]
        return nu, nv

    with jax.named_scope("filter"):
        nu, nv = lax.fori_loop(0, NVEC_IDX, filt, (jnp.int32(0), jnp.int32(0)))
    gcu[pl.ds(nu, LANES)] = jnp.full((LANES,), DUMMY_C, jnp.int32)
    gbu[pl.ds(nu, LANES)] = jnp.full((LANES,), DUMMY_B, jnp.int32)
    gcv[pl.ds(nv, LANES)] = jnp.full((LANES,), DUMMY_C, jnp.int32)
    gbv[pl.ds(nv, LANES)] = jnp.full((LANES,), DUMMY_B, jnp.int32)
    nvec_u = (nu + 2 * LANES - 1) >> 4
    nvec_v = (nv + 2 * LANES - 1) >> 4

    def process(cbuf, par, gc, gb, nvec, fetch, mlo, mhi,
                stag, posmap, flush, sem_f, gcnt):
        def memb(i2, lcnt):
            cv2 = gc[pl.ds(i2 * LANES, LANES)]
            bv2 = gb[pl.ds(i2 * LANES, LANES)]
            m = (cv2 >= mlo) & (cv2 < mhi)
            plsc.store_compressed(lc.at[pl.ds(lcnt, LANES)], cv2, mask=m)
            plsc.store_compressed(lb.at[pl.ds(lcnt, LANES)], bv2, mask=m)
            return lcnt + plsc.all_reduce_population_count(m)[0]

        with jax.named_scope("memb"):
            lcnt = lax.fori_loop(0, nvec, memb, jnp.int32(0))
        lc[pl.ds(lcnt, LANES)] = jnp.full((LANES,), fetch, jnp.int32)
        lb[pl.ds(lcnt, LANES)] = jnp.full((LANES,), DUMMY_B, jnp.int32)
        ngr = (lcnt + LANES - 1) >> 4

        def group(g2, gc2):
            cvec = lc[pl.ds(g2 * LANES, LANES)]
            bvec = lb[pl.ds(g2 * LANES, LANES)]
            lvec = cvec - fetch
            slot = gc2 & 1
            wpos = region + gc2 * LANES
            posb[slot, pl.ds(0, LANES)] = wpos + iota
            hp = pltpu.async_copy(posb.at[slot], posmap.at[bvec], sem_p)

            @pl.when(gc2 >= 2)
            def _():
                pltpu.make_async_copy(
                    flush.at[slot],
                    stag.at[pl.ds(region + (gc2 - 2) * LANES, LANES)],
                    sem_f).wait()

            rows = iota
            for j in range(LANES):
                cols = jnp.full((LANES,), lvec[j], jnp.int32)
                lo = plsc.load_gather(cbuf.at[par], [rows, cols])
                hi = plsc.load_gather(cbuf.at[par], [rows + LANES, cols])
                flush[slot, j, pl.ds(0, LANES)] = lo
                flush[slot, j, pl.ds(LANES, LANES)] = hi
            hp.wait()
            pltpu.async_copy(flush.at[slot], stag.at[pl.ds(wpos, LANES)],
                             sem_f)
            return gc2 + 1

        with jax.named_scope("groups"):
            return lax.fori_loop(0, ngr, group, gcnt)

    def chunk_body(c, carry):
        gu, gv = carry
        par = lax.rem(c, 2)
        fetch, mlo, mhi = _memb_bounds(c, seg_lo)
        with jax.named_scope("chunk_wait"):
            _wait_chunk(ut_t, fetch, cbuf_u, par, sem_s)
            _wait_chunk(it_t, fetch, cbuf_v, par, sem_s)
        gu = process(cbuf_u, par, gcu, gbu, nvec_u, fetch, mlo, mhi,
                     stag_u, posmap_u, flush_u, sem_fu, gu)
        gv = process(cbuf_v, par, gcv, gbv, nvec_v, fetch, mlo, mhi,
                     stag_i, posmap_i, flush_v, sem_fv, gv)

        @pl.when(c + 2 < nch)
        def _():
            f2, _, _ = _memb_bounds(c + 2, seg_lo)
            _issue_chunk(ut_t, f2, cbuf_u, par, sem_s)
            _issue_chunk(it_t, f2, cbuf_v, par, sem_s)

        return gu, gv

    gu, gv = lax.fori_loop(0, nch, chunk_body, (jnp.int32(0), jnp.int32(0)))

    for gcnt, flush, stag, sem_f in ((gu, flush_u, stag_u, sem_fu),
                                     (gv, flush_v, stag_i, sem_fv)):
        @pl.when(gcnt >= 2)
        def _(gcnt=gcnt, flush=flush, stag=stag, sem_f=sem_f):
            pltpu.make_async_copy(
                flush.at[gcnt & 1],
                stag.at[pl.ds(region + (gcnt - 2) * LANES, LANES)],
                sem_f).wait()

        @pl.when(gcnt >= 1)
        def _(gcnt=gcnt, flush=flush, stag=stag, sem_f=sem_f):
            pltpu.make_async_copy(
                flush.at[(gcnt - 1) & 1],
                stag.at[pl.ds(region + (gcnt - 1) * LANES, LANES)],
                sem_f).wait()


@functools.partial(
    pl.kernel,
    out_type=(jax.ShapeDtypeStruct((NW * ROWCAP, 128), jnp.float32),
              jax.ShapeDtypeStruct((NW * ROWCAP, 128), jnp.float32),
              jax.ShapeDtypeStruct((POS_N,), jnp.int32),
              jax.ShapeDtypeStruct((POS_N,), jnp.int32)),
    mesh=plsc.VectorSubcoreMesh(core_axis_name="c", subcore_axis_name="s"),
    scratch_types=[
        pltpu.VMEM((BATCH,), jnp.int32),
        pltpu.VMEM((BATCH,), jnp.int32),
        pltpu.VMEM((GCAP,), jnp.int32),
        pltpu.VMEM((GCAP,), jnp.int32),
        pltpu.VMEM((GCAP,), jnp.int32),
        pltpu.VMEM((GCAP,), jnp.int32),
        pltpu.VMEM((LCAP,), jnp.int32),
        pltpu.VMEM((LCAP,), jnp.int32),
        pltpu.VMEM((2, EMBED, CHUNK_W), jnp.float32),
        pltpu.VMEM((2, EMBED, CHUNK_W), jnp.float32),
        pltpu.VMEM((2, LANES, 128), jnp.float32),
        pltpu.VMEM((2, LANES, 128), jnp.float32),
        pltpu.VMEM((2, LANES), jnp.int32),
        pltpu.SemaphoreType.DMA,
        pltpu.SemaphoreType.DMA,
        pltpu.SemaphoreType.DMA,
        pltpu.SemaphoreType.DMA,
    ],
    compiler_params=pltpu.CompilerParams(needs_layout_passes=False),
)
def _phase_a(user_hbm, item_hbm, ut_t, it_t,
             stag_u, stag_i, posmap_u, posmap_i, *scratch):
    _phase_a_body(user_hbm, item_hbm, ut_t, it_t,
                  stag_u, stag_i, posmap_u, posmap_i, *scratch)


RB_CH = 4
RB_W = B_PER_W // RB_CH


def _phase_b_body(stag_u, stag_i, posmap_u, posmap_i, out_hbm,
                  pos_u, pos_v, sbu, sbv, out_v, sem):
    core = lax.axis_index("c")
    sub = lax.axis_index("s")
    wid = sub * NUM_CORES + core
    base = wid * B_PER_W

    pltpu.sync_copy(posmap_u.at[pl.ds(base, B_PER_W)], pos_u)
    pltpu.sync_copy(posmap_i.at[pl.ds(base, B_PER_W)], pos_v)

    handles = [None] * (RB_CH + 1)
    handles[0] = (
        pltpu.async_copy(stag_u.at[pos_u.at[pl.ds(0, RB_W)]], sbu.at[0], sem),
        pltpu.async_copy(stag_i.at[pos_v.at[pl.ds(0, RB_W)]], sbv.at[0], sem),
    )
    last_lane = lax.iota(jnp.int32, LANES) == (LANES - 1)

    for c in range(RB_CH):
        par = c % 2
        if c + 1 < RB_CH:
            nxt = (c + 1) % 2
            handles[c + 1] = (
                pltpu.async_copy(
                    stag_u.at[pos_u.at[pl.ds((c + 1) * RB_W, RB_W)]],
                    sbu.at[nxt], sem),
                pltpu.async_copy(
                    stag_i.at[pos_v.at[pl.ds((c + 1) * RB_W, RB_W)]],
                    sbv.at[nxt], sem),
            )
        hu, hv = handles[c]
        hu.wait()
        hv.wait()

        def row(r, carry2, c=c, par=par):
            u0 = sbu[par, r, pl.ds(0, LANES)]
            u1 = sbu[par, r, pl.ds(LANES, LANES)]
            v0 = sbv[par, r, pl.ds(0, LANES)]
            v1 = sbv[par, r, pl.ds(LANES, LANES)]
            cs = plsc.cumsum(u0 * v0 + u1 * v1)
            plsc.store_scatter(out_v,
                               [jnp.full((LANES,), c * RB_W + r, jnp.int32)],
                               cs, mask=last_lane)
            return carry2

        lax.fori_loop(0, RB_W, row, 0)

    pltpu.sync_copy(out_v, out_hbm.at[pl.ds(base, B_PER_W)])


@functools.partial(
    pl.kernel,
    out_type=jax.ShapeDtypeStruct((BATCH,), jnp.float32),
    mesh=plsc.VectorSubcoreMesh(core_axis_name="c", subcore_axis_name="s"),
    scratch_types=[
        pltpu.VMEM((B_PER_W,), jnp.int32),
        pltpu.VMEM((B_PER_W,), jnp.int32),
        pltpu.VMEM((2, RB_W, 128), jnp.float32),
        pltpu.VMEM((2, RB_W, 128), jnp.float32),
        pltpu.VMEM((B_PER_W,), jnp.float32),
        pltpu.SemaphoreType.DMA,
    ],
    compiler_params=pltpu.CompilerParams(needs_layout_passes=False),
)
def _phase_b(stag_u, stag_i, posmap_u, posmap_i, out_hbm, *scratch):
    _phase_b_body(stag_u, stag_i, posmap_u, posmap_i, out_hbm, *scratch)


def kernel(user, item, user_table, item_table):
    su, si, pu, pv = _phase_a(user.astype(jnp.int32), item.astype(jnp.int32),
                              user_table.T, item_table.T)
    return _phase_b(su, si, pu, pv)


# speedup vs baseline: 35.3097x; 35.1102x over previous
import functools

import jax
import jax.numpy as jnp
from jax import lax
from jax.experimental import pallas as pl
from jax.experimental.pallas import tpu as pltpu
from jax.experimental.pallas import tpu_sc as plsc

NUM_CORES = 2
NUM_SUBCORES = 16
NW = NUM_CORES * NUM_SUBCORES
BATCH = 16384
B_PER_W = BATCH // NW
EMBED = 32
LANES = 16
DEPTH = 8
GROUPS = B_PER_W // LANES


def _off(c):
    return pl.multiple_of((c >> 7) * 128, 128)


def _issue(tab, c, win, slot, sem):
    pltpu.async_copy(tab.at[:, pl.ds(_off(c), 128)], win.at[slot], sem)


def _retire(tab, c, win, slot, sem):
    pltpu.make_async_copy(tab.at[:, pl.ds(_off(c), 128)],
                          win.at[slot], sem).wait()


def _extract(win, slot, c, res, kp):
    rows = lax.iota(jnp.int32, LANES)
    cols = jnp.full((LANES,), c & 127, jnp.int32)
    kcols = jnp.full((LANES,), kp, jnp.int32)
    lo = plsc.load_gather(win.at[slot], [rows, cols])
    hi = plsc.load_gather(win.at[slot], [rows + LANES, cols])
    plsc.store_scatter(res, [rows, kcols], lo)
    plsc.store_scatter(res, [rows + LANES, kcols], hi)


def _sc_body(user_hbm, item_hbm, ut_t, it_t, out_hbm,
             uidx_v, iidx_v, uwin, iwin, ures, ires, out_v, sem):
    core = lax.axis_index("c")
    sub = lax.axis_index("s")
    wid = sub * NUM_CORES + core
    base = wid * B_PER_W

    pltpu.sync_copy(user_hbm.at[pl.ds(base, B_PER_W)], uidx_v)
    pltpu.sync_copy(item_hbm.at[pl.ds(base, B_PER_W)], iidx_v)

    def step(cu, cv, cu_prev, cv_prev, g, j):
        if j >= DEPTH:
            cpu, cpv = cu[j - DEPTH], cv[j - DEPTH]
        else:
            cpu, cpv = cu_prev[LANES - DEPTH + j], cv_prev[LANES - DEPTH + j]
        slot = (j - DEPTH) % DEPTH
        kp = g * LANES + j - DEPTH
        _retire(ut_t, cpu, uwin, slot, sem)
        _retire(it_t, cpv, iwin, slot, sem)
        _extract(uwin, slot, cpu, ures, kp)
        _extract(iwin, slot, cpv, ires, kp)
        _issue(ut_t, cu[j], uwin, j % DEPTH, sem)
        _issue(it_t, cv[j], iwin, j % DEPTH, sem)

    cu0 = uidx_v[pl.ds(0, LANES)]
    cv0 = iidx_v[pl.ds(0, LANES)]
    for j in range(DEPTH):
        _issue(ut_t, cu0[j], uwin, j, sem)
        _issue(it_t, cv0[j], iwin, j, sem)
    for j in range(DEPTH, LANES):
        step(cu0, cv0, cu0, cv0, 0, j)

    def body(g, carry):
        cu_prev, cv_prev = carry
        cu = uidx_v[pl.ds(g * LANES, LANES)]
        cv = iidx_v[pl.ds(g * LANES, LANES)]
        for j in range(LANES):
            step(cu, cv, cu_prev, cv_prev, g, j)
        return (cu, cv)

    cu_last, cv_last = lax.fori_loop(1, GROUPS, body, (cu0, cv0))

    for j in range(DEPTH):
        cpu = cu_last[LANES - DEPTH + j]
        cpv = cv_last[LANES - DEPTH + j]
        slot = j % DEPTH
        kp = B_PER_W - DEPTH + j
        _retire(ut_t, cpu, uwin, slot, sem)
        _retire(it_t, cpv, iwin, slot, sem)
        _extract(uwin, slot, cpu, ures, kp)
        _extract(iwin, slot, cpv, ires, kp)

    def dot_body(g, carry):
        b0 = g * LANES
        acc = ures[0, pl.ds(b0, LANES)] * ires[0, pl.ds(b0, LANES)]
        for d in range(1, EMBED):
            acc = acc + ures[d, pl.ds(b0, LANES)] * ires[d, pl.ds(b0, LANES)]
        out_v[pl.ds(b0, LANES)] = acc
        return carry

    lax.fori_loop(0, GROUPS, dot_body, 0)

    pltpu.sync_copy(out_v, out_hbm.at[pl.ds(base, B_PER_W)])


@functools.partial(
    pl.kernel,
    out_type=jax.ShapeDtypeStruct((BATCH,), jnp.float32),
    mesh=plsc.VectorSubcoreMesh(core_axis_name="c", subcore_axis_name="s"),
    scratch_types=[
        pltpu.VMEM((B_PER_W,), jnp.int32),
        pltpu.VMEM((B_PER_W,), jnp.int32),
        pltpu.VMEM((DEPTH, EMBED, 128), jnp.float32),
        pltpu.VMEM((DEPTH, EMBED, 128), jnp.float32),
        pltpu.VMEM((EMBED, B_PER_W), jnp.float32),
        pltpu.VMEM((EMBED, B_PER_W), jnp.float32),
        pltpu.VMEM((B_PER_W,), jnp.float32),
        pltpu.SemaphoreType.DMA,
    ],
    compiler_params=pltpu.CompilerParams(needs_layout_passes=False),
)
def _sc_kernel(user_hbm, item_hbm, ut_t, it_t, out_hbm, *scratch):
    _sc_body(user_hbm, item_hbm, ut_t, it_t, out_hbm, *scratch)


def kernel(user, item, user_table, item_table):
    return _sc_kernel(user.astype(jnp.int32), item.astype(jnp.int32),
                      user_table.T, item_table.T)
